# baseline (speedup 1.0000x reference)
import functools

import jax
import jax.numpy as jnp
from jax import lax
from jax.experimental import pallas as pl
from jax.experimental.pallas import tpu as pltpu
from jax.experimental.pallas import tpu_sc as plsc

NUM_CORES = 2
NUM_SUBCORES = 16
LANES = 16
DPAD = 128


def _emb_half(inputs_flat, tab128, pos_packed, B, S, D):
    NW = NUM_CORES * NUM_SUBCORES
    seqs_per_w = B // NW
    idx_per_w = seqs_per_w * S
    SP = S // 2

    mesh = plsc.VectorSubcoreMesh(core_axis_name="c", subcore_axis_name="s")

    @functools.partial(
        pl.kernel,
        mesh=mesh,
        out_type=jax.ShapeDtypeStruct((B, SP, 2 * D), jnp.float32),
        scratch_types=[
            pltpu.VMEM((idx_per_w,), jnp.int32),
            pltpu.VMEM((2, S, DPAD), jnp.float32),
            pltpu.VMEM((2, SP, 2 * D), jnp.float32),
            pltpu.VMEM((SP, 2 * D), jnp.float32),
        ]
        + [pltpu.SemaphoreType.DMA] * 4,
    )
    def emb_kernel(inp_hbm, tab_hbm, pos_hbm, out_hbm, idx_v, rows_v, st_v, pos_v, *sems):
        gsem = sems[:2]
        ssem = sems[2:]
        wid = lax.axis_index("s") * NUM_CORES + lax.axis_index("c")
        base_idx = wid * idx_per_w
        base_seq = wid * seqs_per_w

        pltpu.sync_copy(pos_hbm, pos_v)
        pltpu.sync_copy(
            inp_hbm.at[pl.ds(pl.multiple_of(base_idx, 128), idx_per_w)], idx_v
        )

        def fire(s, b):
            off = pl.multiple_of(s * S, 8)
            pltpu.async_copy(
                tab_hbm.at[idx_v.at[pl.ds(off, 128)]],
                rows_v.at[b, pl.ds(0, 128)],
                gsem[b],
            )
            pltpu.async_copy(
                tab_hbm.at[idx_v.at[pl.ds(off + 128, S - 128)]],
                rows_v.at[b, pl.ds(128, S - 128)],
                gsem[b],
            )

        def wait_sc(b):
            pltpu.make_async_copy(st_v.at[b], out_hbm.at[0], ssem[b]).wait()

        def process(s, b):
            pltpu.make_async_copy(
                tab_hbm.at[pl.ds(0, S)], rows_v.at[b], gsem[b]
            ).wait()

            @plsc.parallel_loop(0, SP, unroll=2)
            def add_body(r2):
                for half in range(2):
                    for ci in range(D // LANES):
                        dsl = pl.ds(half * D + ci * LANES, LANES)
                        ssl = pl.ds(ci * LANES, LANES)
                        st_v[b, r2, dsl] = rows_v[b, 2 * r2 + half, ssl] + pos_v[r2, dsl]
            pltpu.async_copy(st_v.at[b], out_hbm.at[base_seq + s], ssem[b])

        fire(0, 0)
        fire(1, 1)
        process(0, 0)
        fire(2, 0)
        process(1, 1)

        def super_body(k, carry):
            s = 2 * k
            fire(s + 1, 1)
            wait_sc(0)
            process(s, 0)
            fire(s + 2, 0)
            wait_sc(1)
            process(s + 1, 1)
            return carry

        lax.fori_loop(1, seqs_per_w // 2 - 1, super_body, 0)

        s = seqs_per_w - 2
        fire(s + 1, 1)
        wait_sc(0)
        process(s, 0)
        wait_sc(1)
        process(s + 1, 1)
        wait_sc(0)
        wait_sc(1)

    return emb_kernel(inputs_flat, tab128, pos_packed)


def kernel(inputs, token_table, pos_table):
    B, S = inputs.shape
    V, D = token_table.shape
    SP = S // 2
    HB = B // 2

    tab128 = jnp.pad(token_table, ((0, 0), (0, DPAD - D)))
    pos_packed = pos_table.reshape(SP, 2 * D)
    flat = inputs.reshape(B * S)

    h0 = _emb_half(flat[: HB * S], tab128, pos_packed, HB, S, D)
    h1 = _emb_half(flat[HB * S:], tab128, pos_packed, HB, S, D)
    out = jnp.zeros((B, S, D), jnp.float32)
    out = lax.dynamic_update_slice(out, h0.reshape(HB, S, D), (0, 0, 0))
    out = lax.dynamic_update_slice(out, h1.reshape(HB, S, D), (HB, 0, 0))
    return out


# speedup vs baseline: 1.3040x; 1.3040x over previous
import functools

import jax
import jax.numpy as jnp
from jax import lax
from jax.experimental import pallas as pl
from jax.experimental.pallas import tpu as pltpu
from jax.experimental.pallas import tpu_sc as plsc

NUM_CORES = 2
NUM_SUBCORES = 16
LANES = 16
DPAD = 128


def kernel(inputs, token_table, pos_table):
    B, S = inputs.shape
    V, D = token_table.shape
    NW = NUM_CORES * NUM_SUBCORES
    seqs_per_w = B // NW
    idx_per_w = seqs_per_w * S
    SP = S // 2

    tab128 = jnp.pad(token_table, ((0, 0), (0, DPAD - D)))
    pos_packed = pos_table.reshape(SP, 2 * D)
    inputs_flat = inputs.reshape(B * S)

    mesh = plsc.VectorSubcoreMesh(core_axis_name="c", subcore_axis_name="s")

    @functools.partial(
        pl.kernel,
        mesh=mesh,
        out_type=jax.ShapeDtypeStruct((B, SP, 2 * D), jnp.float32),
        scratch_types=[
            pltpu.VMEM((idx_per_w,), jnp.int32),
            pltpu.VMEM((2, S, DPAD), jnp.float32),
            pltpu.VMEM((2, SP, 2 * D), jnp.float32),
            pltpu.VMEM((SP, 2 * D), jnp.float32),
        ]
        + [pltpu.SemaphoreType.DMA] * 4,
    )
    def emb_kernel(inp_hbm, tab_hbm, pos_hbm, out_hbm, idx_v, rows_v, st_v, pos_v, *sems):
        gsem = sems[:2]
        ssem = sems[2:]
        wid = lax.axis_index("s") * NUM_CORES + lax.axis_index("c")
        base_idx = wid * idx_per_w
        base_seq = wid * seqs_per_w

        pltpu.sync_copy(pos_hbm, pos_v)
        pltpu.sync_copy(
            inp_hbm.at[pl.ds(pl.multiple_of(base_idx, 128), idx_per_w)], idx_v
        )

        def fire(s, b):
            off = pl.multiple_of(s * S, 8)
            pltpu.async_copy(
                tab_hbm.at[idx_v.at[pl.ds(off, 128)]],
                rows_v.at[b, pl.ds(0, 128)],
                gsem[b],
            )
            pltpu.async_copy(
                tab_hbm.at[idx_v.at[pl.ds(off + 128, S - 128)]],
                rows_v.at[b, pl.ds(128, S - 128)],
                gsem[b],
            )

        def wait_sc(b):
            pltpu.make_async_copy(st_v.at[b], out_hbm.at[0], ssem[b]).wait()

        def process(s, b):
            pltpu.make_async_copy(
                tab_hbm.at[pl.ds(0, S)], rows_v.at[b], gsem[b]
            ).wait()

            @plsc.parallel_loop(0, SP, unroll=2)
            def add_body(r2):
                for half in range(2):
                    for ci in range(D // LANES):
                        dsl = pl.ds(half * D + ci * LANES, LANES)
                        ssl = pl.ds(ci * LANES, LANES)
                        st_v[b, r2, dsl] = rows_v[b, 2 * r2 + half, ssl] + pos_v[r2, dsl]
            pltpu.async_copy(st_v.at[b], out_hbm.at[base_seq + s], ssem[b])

        fire(0, 0)
        fire(1, 1)
        process(0, 0)
        fire(2, 0)
        process(1, 1)

        def super_body(k, carry):
            s = 2 * k
            fire(s + 1, 1)
            wait_sc(0)
            process(s, 0)
            fire(s + 2, 0)
            wait_sc(1)
            process(s + 1, 1)
            return carry

        lax.fori_loop(1, seqs_per_w // 2 - 1, super_body, 0)

        s = seqs_per_w - 2
        fire(s + 1, 1)
        wait_sc(0)
        process(s, 0)
        wait_sc(1)
        process(s + 1, 1)
        wait_sc(0)
        wait_sc(1)

    out = emb_kernel(inputs_flat, tab128, pos_packed)
    return out.reshape(B, S, D)
